# Initial kernel scaffold; baseline (speedup 1.0000x reference)
import jax
import jax.numpy as jnp
from jax.experimental import pallas as pl


def kernel(edge_index, edge_values, user_emb, item_emb):
    raise NotImplementedError("write your pallas kernel here")


# speedup vs baseline: 2.3051x; 2.3051x over previous
import functools

import jax
import jax.numpy as jnp
from jax import lax
from jax.experimental import pallas as pl
from jax.experimental.pallas import tpu as pltpu
from jax.experimental.pallas import tpu_sc as plsc

NU = 25000
NI = 25000
D = 64
NL = 3
PAD = 88
HP = NU + PAD
NP = 2 * HP
E = 800000
NTILE = 16
BLK = 128
CHUNK_BLKS = 8
BLOCKS_PER_TILE = 392
EPT = BLOCKS_PER_TILE * BLK
EPAD = EPT * NTILE
NCHUNK = BLOCKS_PER_TILE // CHUNK_BLKS
ROWS_PER_TILE = HP // NTILE
ZR = 98
NZ = ROWS_PER_TILE // ZR


def _layer(emb, colv, idxv, wv):
    @functools.partial(
        pl.kernel,
        out_type=jax.ShapeDtypeStruct((NP, D), jnp.float32),
        mesh=plsc.VectorSubcoreMesh(
            core_axis_name="c", subcore_axis_name="s",
            num_cores=2, num_subcores=16),
        compiler_params=pltpu.CompilerParams(use_tc_tiling_on_sc=False),
        scratch_types=[
            pltpu.VMEM((CHUNK_BLKS, BLK), jnp.int32),
            pltpu.VMEM((CHUNK_BLKS, BLK), jnp.int32),
            pltpu.VMEM((CHUNK_BLKS * BLK,), jnp.float32),
            pltpu.VMEM((BLK, D), jnp.float32),
            pltpu.VMEM((ZR, D), jnp.float32),
            pltpu.VMEM_SHARED((HP, D), jnp.float32),
            pltpu.SemaphoreType.DMA,
        ],
    )
    def k(emb_hbm, col_hbm, idx_hbm, w_hbm, out_hbm,
          cbuf, dbuf, wbuf, gbuf, zbuf, acc, sem):
        c = lax.axis_index("c")
        s = lax.axis_index("s")

        zeros16 = jnp.zeros((16,), jnp.float32)

        def zb(r, _):
            for dd in range(4):
                zbuf[r, pl.ds(dd * 16, 16)] = zeros16
            return 0

        lax.fori_loop(0, ZR, zb, 0)
        for q in range(NZ):
            pltpu.sync_copy(
                zbuf, acc.at[pl.ds(s * ROWS_PER_TILE + q * ZR, ZR)])
        plsc.subcore_barrier()

        def chunk_body(kk, _):
            row0 = s * BLOCKS_PER_TILE + kk * CHUNK_BLKS
            pltpu.sync_copy(col_hbm.at[pl.ds(row0, CHUNK_BLKS)], cbuf)
            pltpu.sync_copy(idx_hbm.at[c, pl.ds(row0, CHUNK_BLKS)], dbuf)
            pltpu.sync_copy(w_hbm.at[pl.ds(row0 * BLK, CHUNK_BLKS * BLK)], wbuf)
            for b in range(CHUNK_BLKS):
                pltpu.async_copy(emb_hbm.at[cbuf.at[b]], gbuf, sem).wait()

                def scale(g, _):
                    w16 = wbuf[pl.ds(b * BLK + g * 16, 16)]
                    for j in range(16):
                        ws = jnp.broadcast_to(w16[j], (16,))
                        e = g * 16 + j
                        for dd in range(4):
                            sl = pl.ds(dd * 16, 16)
                            gbuf[e, sl] = gbuf[e, sl] * ws
                    return 0

                lax.fori_loop(0, BLK // 16, scale, 0)
                pltpu.sync_copy(gbuf, acc.at[dbuf.at[b]], add=True)
            return 0

        lax.fori_loop(0, NCHUNK, chunk_body, 0)
        plsc.subcore_barrier()

        for q in range(NZ):
            off = s * ROWS_PER_TILE + q * ZR
            pltpu.sync_copy(acc.at[pl.ds(off, ZR)], zbuf)
            pltpu.sync_copy(zbuf, out_hbm.at[pl.ds(c * HP + off, ZR)])

    return k(emb, colv, idxv, wv)


def _mean4(e0, e1, e2, e3):
    M = NP * D // 128
    BM = M // 8

    def mk(x0, x1, x2, x3, o):
        o[...] = (x0[...] + x1[...] + x2[...] + x3[...]) * 0.25

    out = pl.pallas_call(
        mk,
        out_shape=jax.ShapeDtypeStruct((M, 128), jnp.float32),
        grid=(8,),
        in_specs=[pl.BlockSpec((BM, 128), lambda i: (i, 0))] * 4,
        out_specs=pl.BlockSpec((BM, 128), lambda i: (i, 0)),
    )(e0.reshape(M, 128), e1.reshape(M, 128),
      e2.reshape(M, 128), e3.reshape(M, 128))
    return out.reshape(NP, D)


def kernel(edge_index, edge_values, user_emb, item_emb):
    row = edge_index[0]
    col = edge_index[1]

    colp = col + jnp.where(col >= NU, PAD, 0).astype(jnp.int32)
    e_ar = jnp.arange(E, dtype=jnp.int32)
    trash = NU + (e_ar % PAD)
    idx_halves = []
    for cix in range(2):
        local = row - cix * NU
        inb = (local >= 0) & (local < NU)
        idx_halves.append(jnp.where(inb, local, trash).astype(jnp.int32))
    idxv = jnp.stack(idx_halves)

    padn = EPAD - E
    colp = jnp.pad(colp, (0, padn))
    idxv = jnp.pad(idxv, ((0, 0), (0, padn)), constant_values=NU)
    wv = jnp.pad(edge_values, (0, padn))

    colv = colp.reshape(EPAD // BLK, BLK)
    idxv = idxv.reshape(2, EPAD // BLK, BLK)
    wv = wv.reshape(EPAD)

    emb0 = jnp.concatenate([
        user_emb, jnp.zeros((PAD, D), jnp.float32),
        item_emb, jnp.zeros((PAD, D), jnp.float32)], axis=0)

    e1 = _layer(emb0, colv, idxv, wv)
    e2 = _layer(e1, colv, idxv, wv)
    e3 = _layer(e2, colv, idxv, wv)
    out = _mean4(emb0, e1, e2, e3)

    users = out[:NU]
    items = out[HP:HP + NI]
    return users, items


# speedup vs baseline: 7.5148x; 3.2601x over previous
import functools

import jax
import jax.numpy as jnp
from jax import lax
from jax.experimental import pallas as pl
from jax.experimental.pallas import tpu as pltpu
from jax.experimental.pallas import tpu_sc as plsc

NU = 25000
NI = 25000
N = NU + NI
D = 64
DH = 32
NP = 50176
E = 800000
NTILE = 16
BLK = 128
CHUNK_BLKS = 8
BLOCKS_PER_TILE = 392
EPT = BLOCKS_PER_TILE * BLK
EPAD = EPT * NTILE
NCHUNK = BLOCKS_PER_TILE // CHUNK_BLKS
ROWS_PER_TILE = NP // NTILE
ZR = 196
NZ = ROWS_PER_TILE // ZR


def _layer(emb2, colv, idxv, wv):
    @functools.partial(
        pl.kernel,
        out_type=jax.ShapeDtypeStruct((2, NP, DH), jnp.float32),
        mesh=plsc.VectorSubcoreMesh(
            core_axis_name="c", subcore_axis_name="s",
            num_cores=2, num_subcores=16),
        compiler_params=pltpu.CompilerParams(use_tc_tiling_on_sc=False),
        scratch_types=[
            pltpu.VMEM((2, CHUNK_BLKS, BLK), jnp.int32),
            pltpu.VMEM((2, CHUNK_BLKS, BLK), jnp.int32),
            pltpu.VMEM((2, CHUNK_BLKS * BLK), jnp.float32),
            pltpu.VMEM((BLK, DH), jnp.float32),
            pltpu.VMEM((BLK, DH), jnp.float32),
            pltpu.VMEM((ZR, DH), jnp.float32),
            pltpu.VMEM_SHARED((NP, DH), jnp.float32),
            pltpu.SemaphoreType.DMA,
            pltpu.SemaphoreType.DMA,
            pltpu.SemaphoreType.DMA,
        ],
    )
    def k(emb_hbm, col_hbm, idx_hbm, w_hbm, out_hbm,
          cbuf, dbuf, wbuf, gbuf0, gbuf1, zbuf, acc, gsem, ssem, isem):
        c = lax.axis_index("c")
        s = lax.axis_index("s")

        def idx_load(slot, kk):
            row0 = s * BLOCKS_PER_TILE + kk * CHUNK_BLKS
            pltpu.async_copy(
                col_hbm.at[pl.ds(row0, CHUNK_BLKS)], cbuf.at[slot], isem)
            pltpu.async_copy(
                idx_hbm.at[pl.ds(row0, CHUNK_BLKS)], dbuf.at[slot], isem)
            pltpu.async_copy(
                w_hbm.at[pl.ds(row0 * BLK, CHUNK_BLKS * BLK)],
                wbuf.at[slot], isem)

        def idx_wait():
            pltpu.make_async_copy(
                col_hbm.at[pl.ds(0, CHUNK_BLKS)], cbuf.at[0], isem).wait()
            pltpu.make_async_copy(
                idx_hbm.at[pl.ds(0, CHUNK_BLKS)], dbuf.at[0], isem).wait()
            pltpu.make_async_copy(
                w_hbm.at[pl.ds(0, CHUNK_BLKS * BLK)], wbuf.at[0], isem).wait()

        gsrc = emb_hbm.at[c]

        idx_load(0, 0)

        zeros16 = jnp.zeros((16,), jnp.float32)

        def zb(r, _):
            for dd in range(DH // 16):
                zbuf[r, pl.ds(dd * 16, 16)] = zeros16
            return 0

        lax.fori_loop(0, ZR, zb, 0)
        for q in range(NZ):
            pltpu.sync_copy(
                zbuf, acc.at[pl.ds(s * ROWS_PER_TILE + q * ZR, ZR)])
        idx_wait()
        pltpu.async_copy(gsrc.at[cbuf.at[0, 0]], gbuf0, gsem)
        plsc.subcore_barrier()

        def chunk_body(kk, _):
            p = lax.rem(kk, 2)
            pn = 1 - p
            pltpu.make_async_copy(
                gsrc.at[cbuf.at[0, 0]], gbuf0, gsem).wait()
            idx_load(pn, jnp.minimum(kk + 1, NCHUNK - 1))
            bufs = (gbuf0, gbuf1)
            gd = None
            sd = [None] * CHUNK_BLKS
            for b in range(CHUNK_BLKS):
                cur = bufs[b % 2]
                oth = bufs[1 - b % 2]
                if b > 0:
                    gd.wait()
                if b < CHUNK_BLKS - 1:
                    if b >= 1:
                        sd[b - 1].wait()
                    gd = pltpu.async_copy(
                        gsrc.at[cbuf.at[p, b + 1]], oth, gsem)

                def scale(g, _, cur=cur, b=b):
                    w16 = wbuf[p, pl.ds(b * BLK + g * 16, 16)]
                    for q in range(2):
                        e0 = g * 16 + q * 8
                        ws = [jnp.broadcast_to(w16[q * 8 + j], (16,))
                              for j in range(8)]
                        vals = [cur[e0 + j, pl.ds(dd * 16, 16)]
                                for j in range(8) for dd in range(2)]
                        for j in range(8):
                            for dd in range(2):
                                cur[e0 + j, pl.ds(dd * 16, 16)] = (
                                    vals[j * 2 + dd] * ws[j])
                    return 0

                lax.fori_loop(0, BLK // 16, scale, 0)
                sd[b] = pltpu.async_copy(
                    cur, acc.at[dbuf.at[p, b]], ssem, add=True)
            sd[CHUNK_BLKS - 2].wait()
            sd[CHUNK_BLKS - 1].wait()
            idx_wait()
            pltpu.async_copy(gsrc.at[cbuf.at[pn, 0]], gbuf0, gsem)
            return 0

        lax.fori_loop(0, NCHUNK, chunk_body, 0)
        pltpu.make_async_copy(
            gsrc.at[cbuf.at[0, 0]], gbuf0, gsem).wait()
        plsc.subcore_barrier()

        for q in range(NZ):
            off = s * ROWS_PER_TILE + q * ZR
            pltpu.sync_copy(acc.at[pl.ds(off, ZR)], zbuf)
            pltpu.sync_copy(zbuf, out_hbm.at[c, pl.ds(off, ZR)])

    return k(emb2, colv, idxv, wv)


def _mean4(e0, e1, e2, e3):
    M = 2 * NP * DH // 128
    BM = M // 8

    def mk(x0, x1, x2, x3, o):
        o[...] = (x0[...] + x1[...] + x2[...] + x3[...]) * 0.25

    out = pl.pallas_call(
        mk,
        out_shape=jax.ShapeDtypeStruct((M, 128), jnp.float32),
        grid=(8,),
        in_specs=[pl.BlockSpec((BM, 128), lambda i: (i, 0))] * 4,
        out_specs=pl.BlockSpec((BM, 128), lambda i: (i, 0)),
    )(e0.reshape(M, 128), e1.reshape(M, 128),
      e2.reshape(M, 128), e3.reshape(M, 128))
    return out.reshape(2, NP, DH)


def kernel(edge_index, edge_values, user_emb, item_emb):
    row = edge_index[0]
    col = edge_index[1]

    padn = EPAD - E
    colp = jnp.pad(col, (0, padn))
    rowp = jnp.pad(row, (0, padn))
    wv = jnp.pad(edge_values, (0, padn))

    colv = colp.reshape(EPAD // BLK, BLK)
    idxv = rowp.reshape(EPAD // BLK, BLK)
    wv = wv.reshape(EPAD)

    all_emb = jnp.concatenate([user_emb, item_emb], axis=0)
    pad_rows = jnp.zeros((NP - N, D), jnp.float32)
    all_emb = jnp.concatenate([all_emb, pad_rows], axis=0)
    emb2 = jnp.stack([all_emb[:, :DH], all_emb[:, DH:]], axis=0)

    e1 = _layer(emb2, colv, idxv, wv)
    e2 = _layer(e1, colv, idxv, wv)
    e3 = _layer(e2, colv, idxv, wv)
    out = _mean4(emb2, e1, e2, e3)

    full = jnp.concatenate([out[0], out[1]], axis=1)
    users = full[:NU]
    items = full[NU:N]
    return users, items


# speedup vs baseline: 7.6588x; 1.0192x over previous
import functools

import jax
import jax.numpy as jnp
from jax import lax
from jax.experimental import pallas as pl
from jax.experimental.pallas import tpu as pltpu
from jax.experimental.pallas import tpu_sc as plsc

NU = 25000
NI = 25000
N = NU + NI
D = 64
NL = 3
DH = 32
NP = 50176
E = 800000
NTILE = 16
BLK = 128
CHUNK_BLKS = 8
BLOCKS_PER_TILE = 392
EPT = BLOCKS_PER_TILE * BLK
EPAD = EPT * NTILE
NCHUNK = BLOCKS_PER_TILE // CHUNK_BLKS
ROWS_PER_TILE = NP // NTILE
ZR = 196
NZ = ROWS_PER_TILE // ZR


def _forward(emb2, colv, idxv, wv):
    @functools.partial(
        pl.kernel,
        out_type=[jax.ShapeDtypeStruct((2, NP, DH), jnp.float32)] * NL,
        mesh=plsc.VectorSubcoreMesh(
            core_axis_name="c", subcore_axis_name="s",
            num_cores=2, num_subcores=16),
        compiler_params=pltpu.CompilerParams(use_tc_tiling_on_sc=False),
        scratch_types=[
            pltpu.VMEM((2, CHUNK_BLKS, BLK), jnp.int32),
            pltpu.VMEM((2, CHUNK_BLKS, BLK), jnp.int32),
            pltpu.VMEM((2, CHUNK_BLKS * BLK), jnp.float32),
            pltpu.VMEM((BLK, DH), jnp.float32),
            pltpu.VMEM((BLK, DH), jnp.float32),
            pltpu.VMEM((ZR, DH), jnp.float32),
            pltpu.VMEM_SHARED((NP, DH), jnp.float32),
            pltpu.SemaphoreType.DMA,
            pltpu.SemaphoreType.DMA,
            pltpu.SemaphoreType.DMA,
        ],
    )
    def k(emb_hbm, col_hbm, idx_hbm, w_hbm, o1_hbm, o2_hbm, o3_hbm,
          cbuf, dbuf, wbuf, gbuf0, gbuf1, zbuf, acc, gsem, ssem, isem):
        c = lax.axis_index("c")
        s = lax.axis_index("s")

        def idx_load(slot, kk):
            row0 = s * BLOCKS_PER_TILE + kk * CHUNK_BLKS
            pltpu.async_copy(
                col_hbm.at[pl.ds(row0, CHUNK_BLKS)], cbuf.at[slot], isem)
            pltpu.async_copy(
                idx_hbm.at[pl.ds(row0, CHUNK_BLKS)], dbuf.at[slot], isem)
            pltpu.async_copy(
                w_hbm.at[pl.ds(row0 * BLK, CHUNK_BLKS * BLK)],
                wbuf.at[slot], isem)

        def idx_wait():
            pltpu.make_async_copy(
                col_hbm.at[pl.ds(0, CHUNK_BLKS)], cbuf.at[0], isem).wait()
            pltpu.make_async_copy(
                idx_hbm.at[pl.ds(0, CHUNK_BLKS)], dbuf.at[0], isem).wait()
            pltpu.make_async_copy(
                w_hbm.at[pl.ds(0, CHUNK_BLKS * BLK)], wbuf.at[0], isem).wait()

        zeros16 = jnp.zeros((16,), jnp.float32)

        def zb(r, _):
            for dd in range(DH // 16):
                zbuf[r, pl.ds(dd * 16, 16)] = zeros16
            return 0

        def phase(src_hbm, out_hbm):
            gsrc = src_hbm.at[c]

            idx_load(0, 0)
            lax.fori_loop(0, ZR, zb, 0)
            for q in range(NZ):
                pltpu.sync_copy(
                    zbuf, acc.at[pl.ds(s * ROWS_PER_TILE + q * ZR, ZR)])
            idx_wait()
            pltpu.async_copy(gsrc.at[cbuf.at[0, 0]], gbuf0, gsem)
            plsc.subcore_barrier()

            def chunk_body(kk, _):
                p = lax.rem(kk, 2)
                pn = 1 - p
                pltpu.make_async_copy(
                    gsrc.at[cbuf.at[0, 0]], gbuf0, gsem).wait()
                idx_load(pn, jnp.minimum(kk + 1, NCHUNK - 1))
                bufs = (gbuf0, gbuf1)
                gd = None
                sd = [None] * CHUNK_BLKS
                for b in range(CHUNK_BLKS):
                    cur = bufs[b % 2]
                    oth = bufs[1 - b % 2]
                    if b > 0:
                        gd.wait()
                    if b < CHUNK_BLKS - 1:
                        if b >= 1:
                            sd[b - 1].wait()
                        gd = pltpu.async_copy(
                            gsrc.at[cbuf.at[p, b + 1]], oth, gsem)

                    def scale(g, _, cur=cur, b=b, p=p):
                        w16 = wbuf[p, pl.ds(b * BLK + g * 16, 16)]
                        for q in range(2):
                            e0 = g * 16 + q * 8
                            ws = [jnp.broadcast_to(w16[q * 8 + j], (16,))
                                  for j in range(8)]
                            vals = [cur[e0 + j, pl.ds(dd * 16, 16)]
                                    for j in range(8) for dd in range(2)]
                            for j in range(8):
                                for dd in range(2):
                                    cur[e0 + j, pl.ds(dd * 16, 16)] = (
                                        vals[j * 2 + dd] * ws[j])
                        return 0

                    lax.fori_loop(0, BLK // 16, scale, 0)
                    sd[b] = pltpu.async_copy(
                        cur, acc.at[dbuf.at[p, b]], ssem, add=True)
                sd[CHUNK_BLKS - 2].wait()
                sd[CHUNK_BLKS - 1].wait()
                idx_wait()
                pltpu.async_copy(gsrc.at[cbuf.at[pn, 0]], gbuf0, gsem)
                return 0

            lax.fori_loop(0, NCHUNK, chunk_body, 0)
            pltpu.make_async_copy(
                gsrc.at[cbuf.at[0, 0]], gbuf0, gsem).wait()
            plsc.subcore_barrier()

            for q in range(NZ):
                off = s * ROWS_PER_TILE + q * ZR
                pltpu.sync_copy(acc.at[pl.ds(off, ZR)], zbuf)
                pltpu.sync_copy(zbuf, out_hbm.at[c, pl.ds(off, ZR)])
            plsc.subcore_barrier()

        phase(emb_hbm, o1_hbm)
        phase(o1_hbm, o2_hbm)
        phase(o2_hbm, o3_hbm)

    return k(emb2, colv, idxv, wv)


def _mean4(e0, e1, e2, e3):
    M = 2 * NP * DH // 128
    BM = M // 8

    def mk(x0, x1, x2, x3, o):
        o[...] = (x0[...] + x1[...] + x2[...] + x3[...]) * 0.25

    out = pl.pallas_call(
        mk,
        out_shape=jax.ShapeDtypeStruct((M, 128), jnp.float32),
        grid=(8,),
        in_specs=[pl.BlockSpec((BM, 128), lambda i: (i, 0))] * 4,
        out_specs=pl.BlockSpec((BM, 128), lambda i: (i, 0)),
    )(e0.reshape(M, 128), e1.reshape(M, 128),
      e2.reshape(M, 128), e3.reshape(M, 128))
    return out.reshape(2, NP, DH)


def kernel(edge_index, edge_values, user_emb, item_emb):
    row = edge_index[0]
    col = edge_index[1]

    padn = EPAD - E
    colp = jnp.pad(col, (0, padn))
    rowp = jnp.pad(row, (0, padn))
    wv = jnp.pad(edge_values, (0, padn))

    colv = colp.reshape(EPAD // BLK, BLK)
    idxv = rowp.reshape(EPAD // BLK, BLK)
    wv = wv.reshape(EPAD)

    all_emb = jnp.concatenate([user_emb, item_emb], axis=0)
    pad_rows = jnp.zeros((NP - N, D), jnp.float32)
    all_emb = jnp.concatenate([all_emb, pad_rows], axis=0)
    emb2 = jnp.stack([all_emb[:, :DH], all_emb[:, DH:]], axis=0)

    e1, e2, e3 = _forward(emb2, colv, idxv, wv)
    out = _mean4(emb2, e1, e2, e3)

    full = jnp.concatenate([out[0], out[1]], axis=1)
    users = full[:NU]
    items = full[NU:N]
    return users, items


# speedup vs baseline: 10.3641x; 1.3532x over previous
import functools

import jax
import jax.numpy as jnp
from jax import lax
from jax.experimental import pallas as pl
from jax.experimental.pallas import tpu as pltpu
from jax.experimental.pallas import tpu_sc as plsc

NU = 25000
NI = 25000
N = NU + NI
D = 64
NL = 3
DH = 32
NP = 50176
E = 800000
NTILE = 16
BLK = 128
CHUNK_BLKS = 8
BLOCKS_PER_TILE = 392
EPT = BLOCKS_PER_TILE * BLK
EPAD = EPT * NTILE
NCHUNK = BLOCKS_PER_TILE // CHUNK_BLKS
ROWS_PER_TILE = NP // NTILE
ZR = 196
NZ = ROWS_PER_TILE // ZR


def _forward(emb2, colv, idxv, wv):
    @functools.partial(
        pl.kernel,
        out_type=[jax.ShapeDtypeStruct((2, NP, DH), jnp.float32)] * NL,
        mesh=plsc.VectorSubcoreMesh(
            core_axis_name="c", subcore_axis_name="s",
            num_cores=2, num_subcores=16),
        compiler_params=pltpu.CompilerParams(use_tc_tiling_on_sc=False),
        scratch_types=[
            pltpu.VMEM((2, CHUNK_BLKS, BLK), jnp.int32),
            pltpu.VMEM((2, CHUNK_BLKS, BLK), jnp.int32),
            pltpu.VMEM((2, CHUNK_BLKS * BLK), jnp.float32),
            pltpu.VMEM((BLK, DH), jnp.float32),
            pltpu.VMEM((BLK, DH), jnp.float32),
            pltpu.VMEM((BLK, DH), jnp.float32),
            pltpu.VMEM((BLK, DH), jnp.float32),
            pltpu.VMEM((ZR, DH), jnp.float32),
            pltpu.VMEM_SHARED((NP, DH), jnp.float32),
            pltpu.SemaphoreType.DMA,
            pltpu.SemaphoreType.DMA,
            pltpu.SemaphoreType.DMA,
            pltpu.SemaphoreType.DMA,
            pltpu.SemaphoreType.DMA,
        ],
    )
    def k(emb_hbm, col_hbm, idx_hbm, w_hbm, o1_hbm, o2_hbm, o3_hbm,
          cbuf, dbuf, wbuf, gbuf0, gbuf1, gbuf2, gbuf3, zbuf, acc,
          gsem0, gsem1, ssem0, ssem1, isem):
        gsems = (gsem0, gsem1)
        ssems = (ssem0, ssem1)
        c = lax.axis_index("c")
        s = lax.axis_index("s")

        def idx_load(slot, kk):
            row0 = s * BLOCKS_PER_TILE + kk * CHUNK_BLKS
            pltpu.async_copy(
                col_hbm.at[pl.ds(row0, CHUNK_BLKS)], cbuf.at[slot], isem)
            pltpu.async_copy(
                idx_hbm.at[pl.ds(row0, CHUNK_BLKS)], dbuf.at[slot], isem)
            pltpu.async_copy(
                w_hbm.at[pl.ds(row0 * BLK, CHUNK_BLKS * BLK)],
                wbuf.at[slot], isem)

        def idx_wait():
            pltpu.make_async_copy(
                col_hbm.at[pl.ds(0, CHUNK_BLKS)], cbuf.at[0], isem).wait()
            pltpu.make_async_copy(
                idx_hbm.at[pl.ds(0, CHUNK_BLKS)], dbuf.at[0], isem).wait()
            pltpu.make_async_copy(
                w_hbm.at[pl.ds(0, CHUNK_BLKS * BLK)], wbuf.at[0], isem).wait()

        zeros16 = jnp.zeros((16,), jnp.float32)

        def zb(r, _):
            for dd in range(DH // 16):
                zbuf[r, pl.ds(dd * 16, 16)] = zeros16
            return 0

        def phase(src_hbm, out_hbm):
            gsrc = src_hbm.at[c]

            idx_load(0, 0)
            lax.fori_loop(0, ZR, zb, 0)
            for q in range(NZ):
                pltpu.sync_copy(
                    zbuf, acc.at[pl.ds(s * ROWS_PER_TILE + q * ZR, ZR)])
            idx_wait()
            bufs = (gbuf0, gbuf1, gbuf2, gbuf3)
            pltpu.async_copy(gsrc.at[cbuf.at[0, 0]], bufs[0], gsems[0])
            pltpu.async_copy(gsrc.at[cbuf.at[0, 1]], bufs[1], gsems[1])
            plsc.subcore_barrier()

            def chunk_body(kk, _):
                p = lax.rem(kk, 2)
                pn = 1 - p
                idx_load(pn, jnp.minimum(kk + 1, NCHUNK - 1))
                gd = [None] * CHUNK_BLKS
                sd = [None] * CHUNK_BLKS
                for b in range(CHUNK_BLKS):
                    cur = bufs[b % 4]
                    if b < 2:
                        pltpu.make_async_copy(
                            gsrc.at[cbuf.at[0, 0]], cur, gsems[b % 2]).wait()
                    else:
                        gd[b].wait()
                    if b >= 2:
                        sd[b - 2].wait()
                    if b < CHUNK_BLKS - 2:
                        gd[b + 2] = pltpu.async_copy(
                            gsrc.at[cbuf.at[p, b + 2]],
                            bufs[(b + 2) % 4], gsems[b % 2])
                    else:
                        if b == CHUNK_BLKS - 2:
                            idx_wait()
                        pltpu.async_copy(
                            gsrc.at[cbuf.at[pn, b - (CHUNK_BLKS - 2)]],
                            bufs[(b + 2) % 4], gsems[b % 2])

                    def scale(g, _, cur=cur, b=b, p=p):
                        w16 = wbuf[p, pl.ds(b * BLK + g * 16, 16)]
                        for q in range(2):
                            e0 = g * 16 + q * 8
                            ws = [jnp.broadcast_to(w16[q * 8 + j], (16,))
                                  for j in range(8)]
                            vals = [cur[e0 + j, pl.ds(dd * 16, 16)]
                                    for j in range(8) for dd in range(2)]
                            for j in range(8):
                                for dd in range(2):
                                    cur[e0 + j, pl.ds(dd * 16, 16)] = (
                                        vals[j * 2 + dd] * ws[j])
                        return 0

                    lax.fori_loop(0, BLK // 16, scale, 0)
                    sd[b] = pltpu.async_copy(
                        cur, acc.at[dbuf.at[p, b]], ssems[b % 2], add=True)
                sd[CHUNK_BLKS - 2].wait()
                sd[CHUNK_BLKS - 1].wait()
                return 0

            lax.fori_loop(0, NCHUNK, chunk_body, 0)
            pltpu.make_async_copy(
                gsrc.at[cbuf.at[0, 0]], gbuf0, gsems[0]).wait()
            pltpu.make_async_copy(
                gsrc.at[cbuf.at[0, 0]], gbuf1, gsems[1]).wait()
            plsc.subcore_barrier()

            for q in range(NZ):
                off = s * ROWS_PER_TILE + q * ZR
                pltpu.sync_copy(acc.at[pl.ds(off, ZR)], zbuf)
                pltpu.sync_copy(zbuf, out_hbm.at[c, pl.ds(off, ZR)])
            plsc.subcore_barrier()

        phase(emb_hbm, o1_hbm)
        phase(o1_hbm, o2_hbm)
        phase(o2_hbm, o3_hbm)

    return k(emb2, colv, idxv, wv)


def _mean4(e0, e1, e2, e3):
    BM = NP // 8

    def mk(x0, x1, x2, x3, o):
        for h in range(2):
            o[:, pl.ds(h * DH, DH)] = (
                x0[h] + x1[h] + x2[h] + x3[h]) * 0.25

    ispec = pl.BlockSpec((2, BM, DH), lambda i: (0, i, 0))
    return pl.pallas_call(
        mk,
        out_shape=jax.ShapeDtypeStruct((NP, D), jnp.float32),
        grid=(8,),
        in_specs=[ispec] * 4,
        out_specs=pl.BlockSpec((BM, D), lambda i: (i, 0)),
    )(e0, e1, e2, e3)


def kernel(edge_index, edge_values, user_emb, item_emb):
    row = edge_index[0]
    col = edge_index[1]

    padn = EPAD - E
    colp = jnp.pad(col, (0, padn))
    rowp = jnp.pad(row, (0, padn))
    wv = jnp.pad(edge_values, (0, padn))

    colv = colp.reshape(EPAD // BLK, BLK)
    idxv = rowp.reshape(EPAD // BLK, BLK)
    wv = wv.reshape(EPAD)

    all_emb = jnp.concatenate([user_emb, item_emb], axis=0)
    pad_rows = jnp.zeros((NP - N, D), jnp.float32)
    all_emb = jnp.concatenate([all_emb, pad_rows], axis=0)
    emb2 = jnp.stack([all_emb[:, :DH], all_emb[:, DH:]], axis=0)

    e1, e2, e3 = _forward(emb2, colv, idxv, wv)
    full = _mean4(emb2, e1, e2, e3)
    users = full[:NU]
    items = full[NU:N]
    return users, items


# speedup vs baseline: 10.8223x; 1.0442x over previous
import functools

import jax
import jax.numpy as jnp
from jax import lax
from jax.experimental import pallas as pl
from jax.experimental.pallas import tpu as pltpu
from jax.experimental.pallas import tpu_sc as plsc

NU = 25000
NI = 25000
N = NU + NI
D = 64
DH = 32
NP = 50176
E = 800000
NTILE = 16
BLK = 128
CHUNK_BLKS = 8
CHUNK = CHUNK_BLKS * BLK
EPT = E // NTILE
NCHUNK = EPT // CHUNK
TAIL = EPT - NCHUNK * CHUNK
TAIL_G = TAIL // 16
ROWS_PER_TILE = NP // NTILE
ZR = 98
NZ = ROWS_PER_TILE // ZR


def _forward(emb2, eidx, ew):
    @functools.partial(
        pl.kernel,
        out_type=[jax.ShapeDtypeStruct((2, NP, DH), jnp.float32)] * 3,
        mesh=plsc.VectorSubcoreMesh(
            core_axis_name="c", subcore_axis_name="s",
            num_cores=2, num_subcores=16),
        compiler_params=pltpu.CompilerParams(use_tc_tiling_on_sc=False),
        scratch_types=[
            pltpu.VMEM((2, CHUNK), jnp.int32),
            pltpu.VMEM((2, CHUNK), jnp.int32),
            pltpu.VMEM((2, CHUNK_BLKS, BLK), jnp.int32),
            pltpu.VMEM((2, CHUNK_BLKS, BLK), jnp.int32),
            pltpu.VMEM((2, CHUNK), jnp.float32),
            pltpu.VMEM((BLK, DH), jnp.float32),
            pltpu.VMEM((BLK, DH), jnp.float32),
            pltpu.VMEM((BLK, DH), jnp.float32),
            pltpu.VMEM((BLK, DH), jnp.float32),
            pltpu.VMEM_SHARED((NP, DH), jnp.float32),
            pltpu.SemaphoreType.DMA,
            pltpu.SemaphoreType.DMA,
            pltpu.SemaphoreType.DMA,
            pltpu.SemaphoreType.DMA,
            pltpu.SemaphoreType.DMA,
        ],
    )
    def k(emb_hbm, eidx_hbm, ew_hbm, o1_hbm, o2_hbm, o4_hbm,
          cflat, dflat, cbuf, dbuf, wbuf, gbuf0, gbuf1, gbuf2, gbuf3,
          acc, gsem0, gsem1, ssem0, ssem1, isem):
        gsems = (gsem0, gsem1)
        ssems = (ssem0, ssem1)
        bufs = (gbuf0, gbuf1, gbuf2, gbuf3)
        c = lax.axis_index("c")
        s = lax.axis_index("s")
        zeros16 = jnp.zeros((16,), jnp.float32)
        izeros16 = jnp.zeros((16,), jnp.int32)

        def idx_load(slot, kk):
            base = s * EPT + kk * CHUNK
            pltpu.async_copy(
                eidx_hbm.at[1, pl.ds(base, CHUNK)], cflat.at[slot], isem)
            pltpu.async_copy(
                eidx_hbm.at[0, pl.ds(base, CHUNK)], dflat.at[slot], isem)
            pltpu.async_copy(
                ew_hbm.at[pl.ds(base, CHUNK)], wbuf.at[slot], isem)

        def idx_wait():
            pltpu.make_async_copy(
                eidx_hbm.at[0, pl.ds(0, CHUNK)], cflat.at[0], isem).wait()
            pltpu.make_async_copy(
                eidx_hbm.at[0, pl.ds(0, CHUNK)], dflat.at[0], isem).wait()
            pltpu.make_async_copy(
                ew_hbm.at[pl.ds(0, CHUNK)], wbuf.at[0], isem).wait()

        def repack(slot):
            def rp(b, _):
                for g in range(BLK // 16):
                    sl = pl.ds(b * BLK + g * 16, 16)
                    sl2 = pl.ds(g * 16, 16)
                    cbuf[slot, b, sl2] = cflat[slot, sl]
                    dbuf[slot, b, sl2] = dflat[slot, sl]
                return 0

            lax.fori_loop(0, CHUNK_BLKS, rp, 0)

        def scale_block(cur, w_slot, b):
            def scale(g, _):
                w16 = wbuf[w_slot, pl.ds(b * BLK + g * 16, 16)]
                for q in range(2):
                    e0 = g * 16 + q * 8
                    ws = [jnp.broadcast_to(w16[q * 8 + j], (16,))
                          for j in range(8)]
                    vals = [cur[e0 + j, pl.ds(dd * 16, 16)]
                            for j in range(8) for dd in range(2)]
                    for j in range(8):
                        for dd in range(2):
                            cur[e0 + j, pl.ds(dd * 16, 16)] = (
                                vals[j * 2 + dd] * ws[j])
                return 0

            lax.fori_loop(0, BLK // 16, scale, 0)

        def phase(src_hbm, out_hbm):
            gsrc = src_hbm.at[c]

            idx_load(0, 0)

            def zb(r, _):
                for dd in range(DH // 16):
                    gbuf0[r, pl.ds(dd * 16, 16)] = zeros16
                return 0

            lax.fori_loop(0, ZR, zb, 0)
            for q in range(NZ):
                pltpu.sync_copy(
                    gbuf0.at[pl.ds(0, ZR)],
                    acc.at[pl.ds(s * ROWS_PER_TILE + q * ZR, ZR)])
            idx_wait()
            repack(0)
            pltpu.async_copy(gsrc.at[cbuf.at[0, 0]], bufs[0], gsems[0])
            pltpu.async_copy(gsrc.at[cbuf.at[0, 1]], bufs[1], gsems[1])
            plsc.subcore_barrier()

            def chunk_body(kk, _):
                p = lax.rem(kk, 2)
                pn = 1 - p
                idx_load(pn, jnp.minimum(kk + 1, NCHUNK - 1))
                gd = [None] * CHUNK_BLKS
                sd = [None] * CHUNK_BLKS
                for b in range(CHUNK_BLKS):
                    cur = bufs[b % 4]
                    if b < 2:
                        pltpu.make_async_copy(
                            gsrc.at[cbuf.at[0, 0]], cur, gsems[b % 2]).wait()
                    else:
                        gd[b].wait()
                    if b >= 2:
                        sd[b - 2].wait()
                    if b < CHUNK_BLKS - 2:
                        gd[b + 2] = pltpu.async_copy(
                            gsrc.at[cbuf.at[p, b + 2]],
                            bufs[(b + 2) % 4], gsems[b % 2])
                    else:
                        if b == CHUNK_BLKS - 2:
                            idx_wait()
                            repack(pn)
                        pltpu.async_copy(
                            gsrc.at[cbuf.at[pn, b - (CHUNK_BLKS - 2)]],
                            bufs[(b + 2) % 4], gsems[b % 2])
                    scale_block(cur, p, b)
                    sd[b] = pltpu.async_copy(
                        cur, acc.at[dbuf.at[p, b]], ssems[b % 2], add=True)
                sd[CHUNK_BLKS - 2].wait()
                sd[CHUNK_BLKS - 1].wait()
                return 0

            lax.fori_loop(0, NCHUNK, chunk_body, 0)
            pltpu.make_async_copy(
                gsrc.at[cbuf.at[0, 0]], gbuf0, gsems[0]).wait()
            pltpu.make_async_copy(
                gsrc.at[cbuf.at[0, 0]], gbuf1, gsems[1]).wait()

            base = s * EPT + NCHUNK * CHUNK
            pltpu.sync_copy(eidx_hbm.at[1, pl.ds(base, TAIL)],
                            cflat.at[0, pl.ds(0, TAIL)])
            pltpu.sync_copy(eidx_hbm.at[0, pl.ds(base, TAIL)],
                            dflat.at[0, pl.ds(0, TAIL)])
            pltpu.sync_copy(ew_hbm.at[pl.ds(base, TAIL)],
                            wbuf.at[0, pl.ds(0, TAIL)])
            for g in range(TAIL_G, CHUNK // 16):
                sl = pl.ds(g * 16, 16)
                cflat[0, sl] = izeros16
                dflat[0, sl] = izeros16
                wbuf[0, sl] = zeros16
            repack(0)
            sd = [None] * CHUNK_BLKS
            for b in range(CHUNK_BLKS):
                cur = bufs[b % 4]
                if b == 0:
                    pltpu.async_copy(
                        gsrc.at[cbuf.at[0, 0]], cur, gsems[0])
                pltpu.make_async_copy(
                    gsrc.at[cbuf.at[0, 0]], cur, gsems[b % 2]).wait()
                if b < CHUNK_BLKS - 1:
                    if b >= 1:
                        sd[b - 1].wait()
                    pltpu.async_copy(
                        gsrc.at[cbuf.at[0, b + 1]],
                        bufs[(b + 1) % 4], gsems[(b + 1) % 2])
                scale_block(cur, 0, b)
                sd[b] = pltpu.async_copy(
                    cur, acc.at[dbuf.at[0, b]], ssems[b % 2], add=True)
            sd[CHUNK_BLKS - 2].wait()
            sd[CHUNK_BLKS - 1].wait()
            plsc.subcore_barrier()

            if out_hbm is not None:
                for q in range(NZ):
                    off = s * ROWS_PER_TILE + q * ZR
                    pltpu.sync_copy(acc.at[pl.ds(off, ZR)],
                                    gbuf0.at[pl.ds(0, ZR)])
                    pltpu.sync_copy(gbuf0.at[pl.ds(0, ZR)],
                                    out_hbm.at[c, pl.ds(off, ZR)])
                plsc.subcore_barrier()

        phase(emb_hbm, o1_hbm)
        phase(o1_hbm, o2_hbm)
        phase(o2_hbm, None)

        def mean_step(q, _):
            off = s * ROWS_PER_TILE + q * ZR
            d0 = pltpu.async_copy(
                emb_hbm.at[c, pl.ds(off, ZR)], gbuf0.at[pl.ds(0, ZR)], gsem0)
            d1 = pltpu.async_copy(
                o1_hbm.at[c, pl.ds(off, ZR)], gbuf1.at[pl.ds(0, ZR)], gsem1)
            d2 = pltpu.async_copy(
                o2_hbm.at[c, pl.ds(off, ZR)], gbuf2.at[pl.ds(0, ZR)], ssem0)
            d3 = pltpu.async_copy(
                acc.at[pl.ds(off, ZR)], gbuf3.at[pl.ds(0, ZR)], ssem1)
            d0.wait(); d1.wait(); d2.wait(); d3.wait()

            def avg(r, _):
                for dd in range(DH // 16):
                    sl = pl.ds(dd * 16, 16)
                    gbuf0[r, sl] = (
                        (gbuf0[r, sl] + gbuf1[r, sl])
                        + (gbuf2[r, sl] + gbuf3[r, sl])) * 0.25
                return 0

            lax.fori_loop(0, ZR, avg, 0)
            pltpu.sync_copy(gbuf0.at[pl.ds(0, ZR)],
                            o4_hbm.at[c, pl.ds(off, ZR)])
            return 0

        lax.fori_loop(0, NZ, mean_step, 0)

    return k(emb2, eidx, ew)


def _interleave(x):
    BM = NP // 8

    def ik(xb, o):
        for h in range(2):
            o[:, pl.ds(h * DH, DH)] = xb[h]

    return pl.pallas_call(
        ik,
        out_shape=jax.ShapeDtypeStruct((NP, D), jnp.float32),
        grid=(8,),
        in_specs=[pl.BlockSpec((2, BM, DH), lambda i: (0, i, 0))],
        out_specs=pl.BlockSpec((BM, D), lambda i: (i, 0)),
    )(x)


def kernel(edge_index, edge_values, user_emb, item_emb):
    all_emb = jnp.concatenate([
        user_emb, item_emb, jnp.zeros((NP - N, D), jnp.float32)], axis=0)
    emb2 = jnp.stack([all_emb[:, :DH], all_emb[:, DH:]], axis=0)

    o1, o2, o4 = _forward(emb2, edge_index, edge_values)
    del o1, o2
    full = _interleave(o4)
    users = full[:NU]
    items = full[NU:N]
    return users, items


# speedup vs baseline: 10.9631x; 1.0130x over previous
import functools

import jax
import jax.numpy as jnp
from jax import lax
from jax.experimental import pallas as pl
from jax.experimental.pallas import tpu as pltpu
from jax.experimental.pallas import tpu_sc as plsc

NU = 25000
NI = 25000
N = NU + NI
D = 64
DH = 32
NP = 50176
E = 800000
NTILE = 16
BLK = 128
CHUNK_BLKS = 8
CHUNK = CHUNK_BLKS * BLK
EPT = E // NTILE
NCHUNK = EPT // CHUNK
TAIL = EPT - NCHUNK * CHUNK
TAIL_G = TAIL // 16
ROWS_PER_TILE = NP // NTILE
ZR = 98
NZ = ROWS_PER_TILE // ZR


def _forward(emb2, eidx, ew):
    @functools.partial(
        pl.kernel,
        out_type=[jax.ShapeDtypeStruct((2, NP, DH), jnp.float32)] * 3,
        mesh=plsc.VectorSubcoreMesh(
            core_axis_name="c", subcore_axis_name="s",
            num_cores=2, num_subcores=16),
        compiler_params=pltpu.CompilerParams(use_tc_tiling_on_sc=False),
        scratch_types=[
            pltpu.VMEM((2, CHUNK), jnp.int32),
            pltpu.VMEM((2, CHUNK), jnp.int32),
            pltpu.VMEM((2, CHUNK_BLKS, BLK), jnp.int32),
            pltpu.VMEM((2, CHUNK_BLKS, BLK), jnp.int32),
            pltpu.VMEM((2, CHUNK), jnp.float32),
            pltpu.VMEM((BLK, DH), jnp.float32),
            pltpu.VMEM((BLK, DH), jnp.float32),
            pltpu.VMEM((BLK, DH), jnp.float32),
            pltpu.VMEM((BLK, DH), jnp.float32),
            pltpu.VMEM_SHARED((NP, DH), jnp.float32),
            pltpu.SemaphoreType.DMA,
            pltpu.SemaphoreType.DMA,
            pltpu.SemaphoreType.DMA,
            pltpu.SemaphoreType.DMA,
            pltpu.SemaphoreType.DMA,
        ],
    )
    def k(emb_hbm, eidx_hbm, ew_hbm, o1_hbm, o2_hbm, o4_hbm,
          cflat, dflat, cbuf, dbuf, wbuf, gbuf0, gbuf1, gbuf2, gbuf3,
          acc, gsem0, gsem1, ssem0, ssem1, isem):
        gsems = (gsem0, gsem1)
        ssems = (ssem0, ssem1)
        bufs = (gbuf0, gbuf1, gbuf2, gbuf3)
        c = lax.axis_index("c")
        s = lax.axis_index("s")
        zeros16 = jnp.zeros((16,), jnp.float32)
        izeros16 = jnp.zeros((16,), jnp.int32)

        def idx_load(slot, kk):
            base = s * EPT + kk * CHUNK
            pltpu.async_copy(
                eidx_hbm.at[1, pl.ds(base, CHUNK)], cflat.at[slot], isem)
            pltpu.async_copy(
                eidx_hbm.at[0, pl.ds(base, CHUNK)], dflat.at[slot], isem)
            pltpu.async_copy(
                ew_hbm.at[pl.ds(base, CHUNK)], wbuf.at[slot], isem)

        def idx_wait():
            pltpu.make_async_copy(
                eidx_hbm.at[0, pl.ds(0, CHUNK)], cflat.at[0], isem).wait()
            pltpu.make_async_copy(
                eidx_hbm.at[0, pl.ds(0, CHUNK)], dflat.at[0], isem).wait()
            pltpu.make_async_copy(
                ew_hbm.at[pl.ds(0, CHUNK)], wbuf.at[0], isem).wait()

        def repack(slot):
            def rp(b, _):
                for g in range(BLK // 16):
                    sl = pl.ds(b * BLK + g * 16, 16)
                    sl2 = pl.ds(g * 16, 16)
                    cbuf[slot, b, sl2] = cflat[slot, sl]
                    dbuf[slot, b, sl2] = dflat[slot, sl]
                return 0

            lax.fori_loop(0, CHUNK_BLKS, rp, 0)

        def scale_block(cur, w_slot, b):
            def scale(g, _):
                w16 = wbuf[w_slot, pl.ds(b * BLK + g * 16, 16)]
                for q in range(2):
                    e0 = g * 16 + q * 8
                    ws = [jnp.broadcast_to(w16[q * 8 + j], (16,))
                          for j in range(8)]
                    vals = [cur[e0 + j, pl.ds(dd * 16, 16)]
                            for j in range(8) for dd in range(2)]
                    for j in range(8):
                        for dd in range(2):
                            cur[e0 + j, pl.ds(dd * 16, 16)] = (
                                vals[j * 2 + dd] * ws[j])
                return 0

            lax.fori_loop(0, BLK // 16, scale, 0)

        def phase(src_hbm, out_hbm):
            gsrc = src_hbm.at[c]

            idx_load(0, 0)

            def zb(r, _):
                for dd in range(DH // 16):
                    gbuf0[r, pl.ds(dd * 16, 16)] = zeros16
                return 0

            lax.fori_loop(0, ZR, zb, 0)
            for q in range(NZ):
                pltpu.sync_copy(
                    gbuf0.at[pl.ds(0, ZR)],
                    acc.at[pl.ds(s * ROWS_PER_TILE + q * ZR, ZR)])
            idx_wait()
            repack(0)
            pltpu.async_copy(gsrc.at[cbuf.at[0, 0]], bufs[0], gsems[0])
            pltpu.async_copy(gsrc.at[cbuf.at[0, 1]], bufs[1], gsems[1])
            plsc.subcore_barrier()

            def chunk_body(kk, _):
                p = lax.rem(kk, 2)
                pn = 1 - p
                idx_load(pn, jnp.minimum(kk + 1, NCHUNK - 1))
                gd = [None] * CHUNK_BLKS
                sd = [None] * CHUNK_BLKS
                for b in range(CHUNK_BLKS):
                    cur = bufs[b % 4]
                    if b < 2:
                        pltpu.make_async_copy(
                            gsrc.at[cbuf.at[0, 0]], cur, gsems[b % 2]).wait()
                    else:
                        gd[b].wait()
                    if b >= 2:
                        sd[b - 2].wait()
                    if b < CHUNK_BLKS - 2:
                        gd[b + 2] = pltpu.async_copy(
                            gsrc.at[cbuf.at[p, b + 2]],
                            bufs[(b + 2) % 4], gsems[b % 2])
                    else:
                        if b == CHUNK_BLKS - 2:
                            idx_wait()
                            repack(pn)
                        pltpu.async_copy(
                            gsrc.at[cbuf.at[pn, b - (CHUNK_BLKS - 2)]],
                            bufs[(b + 2) % 4], gsems[b % 2])
                    scale_block(cur, p, b)
                    sd[b] = pltpu.async_copy(
                        cur, acc.at[dbuf.at[p, b]], ssems[b % 2], add=True)
                sd[CHUNK_BLKS - 2].wait()
                sd[CHUNK_BLKS - 1].wait()
                return 0

            lax.fori_loop(0, NCHUNK, chunk_body, 0)
            pltpu.make_async_copy(
                gsrc.at[cbuf.at[0, 0]], gbuf0, gsems[0]).wait()
            pltpu.make_async_copy(
                gsrc.at[cbuf.at[0, 0]], gbuf1, gsems[1]).wait()

            base = s * EPT + NCHUNK * CHUNK
            pltpu.sync_copy(eidx_hbm.at[1, pl.ds(base, TAIL)],
                            cflat.at[0, pl.ds(0, TAIL)])
            pltpu.sync_copy(eidx_hbm.at[0, pl.ds(base, TAIL)],
                            dflat.at[0, pl.ds(0, TAIL)])
            pltpu.sync_copy(ew_hbm.at[pl.ds(base, TAIL)],
                            wbuf.at[0, pl.ds(0, TAIL)])
            for g in range(TAIL_G, CHUNK // 16):
                sl = pl.ds(g * 16, 16)
                cflat[0, sl] = izeros16
                dflat[0, sl] = izeros16
                wbuf[0, sl] = zeros16
            repack(0)
            sd = [None] * CHUNK_BLKS
            for b in range(CHUNK_BLKS):
                cur = bufs[b % 4]
                if b == 0:
                    pltpu.async_copy(
                        gsrc.at[cbuf.at[0, 0]], cur, gsems[0])
                pltpu.make_async_copy(
                    gsrc.at[cbuf.at[0, 0]], cur, gsems[b % 2]).wait()
                if b < CHUNK_BLKS - 1:
                    if b >= 1:
                        sd[b - 1].wait()
                    pltpu.async_copy(
                        gsrc.at[cbuf.at[0, b + 1]],
                        bufs[(b + 1) % 4], gsems[(b + 1) % 2])
                scale_block(cur, 0, b)
                sd[b] = pltpu.async_copy(
                    cur, acc.at[dbuf.at[0, b]], ssems[b % 2], add=True)
            sd[CHUNK_BLKS - 2].wait()
            sd[CHUNK_BLKS - 1].wait()
            plsc.subcore_barrier()

            if out_hbm is not None:
                for q in range(NZ):
                    off = s * ROWS_PER_TILE + q * ZR
                    pltpu.sync_copy(acc.at[pl.ds(off, ZR)],
                                    gbuf0.at[pl.ds(0, ZR)])
                    pltpu.sync_copy(gbuf0.at[pl.ds(0, ZR)],
                                    out_hbm.at[c, pl.ds(off, ZR)])
                plsc.subcore_barrier()

        phase(emb_hbm, o1_hbm)
        phase(o1_hbm, o2_hbm)
        phase(o2_hbm, None)

        def mean_step(q, _):
            off = s * ROWS_PER_TILE + q * ZR
            d0 = pltpu.async_copy(
                emb_hbm.at[c, pl.ds(off, ZR)], gbuf0.at[pl.ds(0, ZR)], gsem0)
            d1 = pltpu.async_copy(
                o1_hbm.at[c, pl.ds(off, ZR)], gbuf1.at[pl.ds(0, ZR)], gsem1)
            d2 = pltpu.async_copy(
                o2_hbm.at[c, pl.ds(off, ZR)], gbuf2.at[pl.ds(0, ZR)], ssem0)
            d3 = pltpu.async_copy(
                acc.at[pl.ds(off, ZR)], gbuf3.at[pl.ds(0, ZR)], ssem1)
            d0.wait(); d1.wait(); d2.wait(); d3.wait()

            def avg(r, _):
                for dd in range(DH // 16):
                    sl = pl.ds(dd * 16, 16)
                    gbuf0[r, sl] = (
                        (gbuf0[r, sl] + gbuf1[r, sl])
                        + (gbuf2[r, sl] + gbuf3[r, sl])) * 0.25
                return 0

            lax.fori_loop(0, ZR, avg, 0)
            pltpu.sync_copy(gbuf0.at[pl.ds(0, ZR)],
                            o4_hbm.at[c, pl.ds(off, ZR)])
            return 0

        lax.fori_loop(0, NZ, mean_step, 0)

    return k(emb2, eidx, ew)


def _interleave(x):
    BM = 1000

    def ik(xu, xi, ou, oi):
        for h in range(2):
            ou[:, pl.ds(h * DH, DH)] = xu[h]
            oi[:, pl.ds(h * DH, DH)] = xi[h]

    return pl.pallas_call(
        ik,
        out_shape=[jax.ShapeDtypeStruct((NU, D), jnp.float32),
                   jax.ShapeDtypeStruct((NI, D), jnp.float32)],
        grid=(25,),
        in_specs=[pl.BlockSpec((2, BM, DH), lambda i: (0, i, 0)),
                  pl.BlockSpec((2, BM, DH), lambda i: (0, i + 25, 0))],
        out_specs=[pl.BlockSpec((BM, D), lambda i: (i, 0)),
                   pl.BlockSpec((BM, D), lambda i: (i, 0))],
    )(x, x)


def kernel(edge_index, edge_values, user_emb, item_emb):
    all_emb = jnp.concatenate([
        user_emb, item_emb, jnp.zeros((NP - N, D), jnp.float32)], axis=0)
    emb2 = jnp.stack([all_emb[:, :DH], all_emb[:, DH:]], axis=0)

    o1, o2, o4 = _forward(emb2, edge_index, edge_values)
    del o1, o2
    users, items = _interleave(o4)
    return users, items
